# baseline (speedup 1.0000x reference)
import functools

import jax
import jax.numpy as jnp
from jax import lax
from jax.experimental import pallas as pl
from jax.experimental.pallas import tpu as pltpu
from jax.experimental.pallas import tpu_sc as plsc

_B, _S, _D = 4, 2048, 768
_DFF = 1024
_E = 16
_T = _B * _S
_CAP = 1280
_PAD_ROW = _E * _CAP
_RB = 1280
_NROWS = _PAD_ROW + _RB
_CT = 2048
_G = _T // _CT

_NC, _NS = 2, 16
_NW = _NC * _NS
_TPW = _T // _NW
_CH = 64
_LANES = 16


def _router_body(x_ref, wg_ref, loc0_ref, loc1_ref, w0_ref, w1_ref,
                 counts_ref, carry_ref):
    pid = pl.program_id(0)

    @pl.when(pid == 0)
    def _():
        carry_ref[...] = jnp.zeros_like(carry_ref)

    x = x_ref[...]
    wg = wg_ref[...]
    logits = jnp.dot(x, wg, preferred_element_type=jnp.float32)
    lt = logits.T

    iota_e = lax.broadcasted_iota(jnp.int32, (_E, _CT), 0)
    m1 = jnp.max(lt, axis=0, keepdims=True)
    i1 = jnp.min(jnp.where(lt == m1, iota_e, _E), axis=0, keepdims=True)
    oh1 = iota_e == i1
    masked = jnp.where(oh1, -jnp.inf, lt)
    m2 = jnp.max(masked, axis=0, keepdims=True)
    i2 = jnp.min(jnp.where(masked == m2, iota_e, _E), axis=0, keepdims=True)
    oh2 = iota_e == i2

    g1 = 1.0 / (1.0 + jnp.exp(m2 - m1))
    g2 = 1.0 - g1

    ohsum = (oh1 | oh2).astype(jnp.float32)
    ri = lax.broadcasted_iota(jnp.int32, (_CT, _CT), 0)
    cj = lax.broadcasted_iota(jnp.int32, (_CT, _CT), 1)
    sut = (ri < cj).astype(jnp.float32)
    cex = jnp.dot(ohsum, sut, preferred_element_type=jnp.float32)
    base = carry_ref[0:_E, 0:1]
    p = cex + base
    pos1 = jnp.sum(jnp.where(oh1, p, 0.0), axis=0)
    pos2 = jnp.sum(jnp.where(oh2, p, 0.0), axis=0)
    new_counts = base + jnp.sum(ohsum, axis=1, keepdims=True)
    carry_ref[0:_E, 0:1] = new_counts
    counts_ref[...] = new_counts.astype(jnp.int32)

    keep1 = pos1 < _CAP
    keep2 = pos2 < _CAP
    e1 = i1[0]
    e2 = i2[0]
    loc0_ref[...] = jnp.where(keep1, e1 * _CAP + pos1.astype(jnp.int32),
                              _PAD_ROW)
    loc1_ref[...] = jnp.where(keep2, e2 * _CAP + pos2.astype(jnp.int32),
                              _PAD_ROW)
    w0_ref[...] = jnp.where(keep1, g1[0], 0.0)
    w1_ref[...] = jnp.where(keep2, g2[0], 0.0)


_router = pl.pallas_call(
    _router_body,
    grid=(_G,),
    in_specs=[
        pl.BlockSpec((_CT, _D), lambda i: (i, 0)),
        pl.BlockSpec((_D, _E), lambda i: (0, 0)),
    ],
    out_specs=[
        pl.BlockSpec((_CT,), lambda i: (i,)),
        pl.BlockSpec((_CT,), lambda i: (i,)),
        pl.BlockSpec((_CT,), lambda i: (i,)),
        pl.BlockSpec((_CT,), lambda i: (i,)),
        pl.BlockSpec((_E, 1), lambda i: (0, 0)),
    ],
    out_shape=[
        jax.ShapeDtypeStruct((_T,), jnp.int32),
        jax.ShapeDtypeStruct((_T,), jnp.int32),
        jax.ShapeDtypeStruct((_T,), jnp.float32),
        jax.ShapeDtypeStruct((_T,), jnp.float32),
        jax.ShapeDtypeStruct((_E, 1), jnp.int32),
    ],
    scratch_shapes=[pltpu.VMEM((_E, 128), jnp.float32)],
)


def _dispatch_body(xt_hbm, loc0_hbm, loc1_hbm, buf_hbm,
                   idx0a_v, idx0b_v, idx1a_v, idx1b_v, rows0_v, rows1_v,
                   csem, ssem0, ssem1):
    wid = lax.axis_index("s") * _NC + lax.axis_index("c")
    base = wid * _TPW
    nch = _TPW // _CH
    idx0 = (idx0a_v, idx0b_v)
    idx1 = (idx1a_v, idx1b_v)
    rows = (rows0_v, rows1_v)
    ssem = (ssem0, ssem1)

    def load(ci, p):
        tok0 = base + ci * _CH
        pltpu.sync_copy(loc0_hbm.at[pl.ds(tok0, _CH)], idx0[p])
        pltpu.sync_copy(loc1_hbm.at[pl.ds(tok0, _CH)], idx1[p])
        return pltpu.async_copy(xt_hbm.at[pl.ds(tok0, _CH)], rows[p], csem)

    cp = load(0, 0)
    sc_prev = None
    for ci in range(nch):
        p = ci & 1
        cp.wait()
        s0 = pltpu.async_copy(rows[p], buf_hbm.at[idx0[p]], ssem[p])
        s1 = pltpu.async_copy(rows[p], buf_hbm.at[idx1[p]], ssem[p])
        if sc_prev is not None:
            sc_prev[0].wait()
            sc_prev[1].wait()
        if ci + 1 < nch:
            cp = load(ci + 1, 1 - p)
        sc_prev = (s0, s1)
    sc_prev[0].wait()
    sc_prev[1].wait()


@functools.lru_cache(maxsize=None)
def _get_dispatch():
    return functools.partial(
        pl.kernel,
        out_type=jax.ShapeDtypeStruct((_NROWS, _D), jnp.float32),
        mesh=plsc.VectorSubcoreMesh(core_axis_name="c", subcore_axis_name="s",
                                    num_cores=_NC, num_subcores=_NS),
        scratch_types=[
            pltpu.VMEM((_CH,), jnp.int32),
            pltpu.VMEM((_CH,), jnp.int32),
            pltpu.VMEM((_CH,), jnp.int32),
            pltpu.VMEM((_CH,), jnp.int32),
            pltpu.VMEM((_CH, _D), jnp.float32),
            pltpu.VMEM((_CH, _D), jnp.float32),
            pltpu.SemaphoreType.DMA,
            pltpu.SemaphoreType.DMA,
            pltpu.SemaphoreType.DMA,
        ],
        compiler_params=pltpu.CompilerParams(needs_layout_passes=False),
    )(_dispatch_body)


def _ffn_body(counts_ref, buf_ref, w1_ref, b1_ref, w2_ref, b2_ref, y_ref):
    e = pl.program_id(0)
    i = pl.program_id(1)
    live = jnp.where(e == _E, i == 0,
                     i * _RB < counts_ref[jnp.minimum(e, _E - 1)])

    @pl.when(live)
    def _():
        w1 = w1_ref[0]
        w2 = w2_ref[0]
        hr = _RB // 2
        xa = buf_ref[0:hr, :]
        xb = buf_ref[hr:_RB, :]
        ha = jnp.dot(xa, w1, preferred_element_type=jnp.float32) + b1_ref[0]
        hb = jnp.dot(xb, w1, preferred_element_type=jnp.float32) + b1_ref[0]
        ga = jax.nn.gelu(ha)
        ya = jnp.dot(ga, w2, preferred_element_type=jnp.float32) + b2_ref[0]
        gb = jax.nn.gelu(hb)
        yb = jnp.dot(gb, w2, preferred_element_type=jnp.float32) + b2_ref[0]
        y_ref[0:hr, :] = ya
        y_ref[hr:_RB, :] = yb


_BPE = _CAP // _RB


def _row_block(e, i):
    return jnp.minimum(e * _BPE + i, _NROWS // _RB - 1)


def _wexpert(e):
    return jnp.minimum(e, _E - 1)


_ffn = pl.pallas_call(
    _ffn_body,
    grid_spec=pltpu.PrefetchScalarGridSpec(
        num_scalar_prefetch=1,
        grid=(_E + 1, _BPE),
        in_specs=[
            pl.BlockSpec((_RB, _D), lambda e, i, c: (_row_block(e, i), 0)),
            pl.BlockSpec((1, _D, _DFF), lambda e, i, c: (_wexpert(e), 0, 0)),
            pl.BlockSpec((1, 1, _DFF), lambda e, i, c: (_wexpert(e), 0, 0)),
            pl.BlockSpec((1, _DFF, _D), lambda e, i, c: (_wexpert(e), 0, 0)),
            pl.BlockSpec((1, 1, _D), lambda e, i, c: (_wexpert(e), 0, 0)),
        ],
        out_specs=pl.BlockSpec((_RB, _D), lambda e, i, c: (_row_block(e, i), 0)),
    ),
    out_shape=jax.ShapeDtypeStruct((_NROWS, _D), jnp.float32),
)


_CCH = 32


def _combine_body(y_hbm, loc0_hbm, loc1_hbm, w0_hbm, w1_hbm, out_hbm,
                  idx0a_v, idx0b_v, idx1a_v, idx1b_v,
                  w0a_v, w0b_v, w1a_v, w1b_v,
                  a0_v, a1_v, b0_v, b1_v, gsem0, gsem1, osem):
    wid = lax.axis_index("s") * _NC + lax.axis_index("c")
    base = wid * _TPW
    nch = _TPW // _CCH
    idx0 = (idx0a_v, idx0b_v)
    idx1 = (idx1a_v, idx1b_v)
    w0 = (w0a_v, w0b_v)
    w1 = (w1a_v, w1b_v)
    av = (a0_v, a1_v)
    bv = (b0_v, b1_v)
    gsem = (gsem0, gsem1)

    def load(ci, p):
        tok0 = base + ci * _CCH
        pltpu.sync_copy(loc0_hbm.at[pl.ds(tok0, _CCH)], idx0[p])
        pltpu.sync_copy(loc1_hbm.at[pl.ds(tok0, _CCH)], idx1[p])
        pltpu.sync_copy(w0_hbm.at[pl.ds(tok0, _CCH)], w0[p])
        pltpu.sync_copy(w1_hbm.at[pl.ds(tok0, _CCH)], w1[p])
        ga = pltpu.async_copy(y_hbm.at[idx0[p]], av[p], gsem[p])
        gb = pltpu.async_copy(y_hbm.at[idx1[p]], bv[p], gsem[p])
        return ga, gb

    gs = [load(0, 0), load(1, 1)]
    for ci in range(nch):
        p = ci & 1
        gs[p][0].wait()
        gs[p][1].wait()

        a_v, b_v, w0_v, w1_v = av[p], bv[p], w0[p], w1[p]

        def tbody(t, _):
            tv = jnp.full((_LANES,), t, jnp.int32)
            wv0 = plsc.load_gather(w0_v, [tv])
            wv1 = plsc.load_gather(w1_v, [tv])
            for f in range(_D // _LANES):
                sl = pl.ds(f * _LANES, _LANES)
                a_v[t, sl] = a_v[t, sl] * wv0 + b_v[t, sl] * wv1
            return 0

        lax.fori_loop(0, _CCH, tbody, 0)
        oc = pltpu.async_copy(
            a_v, out_hbm.at[pl.ds(base + ci * _CCH, _CCH)], osem)
        oc.wait()
        if ci + 2 < nch:
            gs[p] = load(ci + 2, p)


@functools.lru_cache(maxsize=None)
def _get_combine():
    return functools.partial(
        pl.kernel,
        out_type=jax.ShapeDtypeStruct((_T, _D), jnp.float32),
        mesh=plsc.VectorSubcoreMesh(core_axis_name="c", subcore_axis_name="s",
                                    num_cores=_NC, num_subcores=_NS),
        scratch_types=(
            [pltpu.VMEM((_CCH,), jnp.int32)] * 4
            + [pltpu.VMEM((_CCH,), jnp.float32)] * 4
            + [pltpu.VMEM((_CCH, _D), jnp.float32)] * 4
            + [pltpu.SemaphoreType.DMA] * 3
        ),
        compiler_params=pltpu.CompilerParams(needs_layout_passes=False),
    )(_combine_body)


def kernel(x, Wg, W1, b1, W2, b2):
    xt = x.reshape(_T, _D)
    loc0, loc1, w0, w1, counts = _router(xt, Wg)
    buf = _get_dispatch()(xt, loc0, loc1)
    y = _ffn(counts.reshape(_E), buf, W1, b1.reshape(_E, 1, _DFF),
             W2, b2.reshape(_E, 1, _D))
    out = _get_combine()(y, loc0, loc1, w0, w1)
    return out.reshape(_B, _S, _D)


# speedup vs baseline: 1.0082x; 1.0082x over previous
import functools

import jax
import jax.numpy as jnp
from jax import lax
from jax.experimental import pallas as pl
from jax.experimental.pallas import tpu as pltpu
from jax.experimental.pallas import tpu_sc as plsc

_B, _S, _D = 4, 2048, 768
_DFF = 1024
_E = 16
_T = _B * _S
_CAP = 1280
_PAD_ROW = _E * _CAP
_RB = 1280
_NROWS = _PAD_ROW + _RB
_CT = 1024
_G = _T // _CT

_NC, _NS = 2, 16
_NW = _NC * _NS
_TPW = _T // _NW
_CH = 64
_LANES = 16


def _router_body(x_ref, wg_ref, loc0_ref, loc1_ref, w0_ref, w1_ref,
                 counts_ref, carry_ref):
    pid = pl.program_id(0)

    @pl.when(pid == 0)
    def _():
        carry_ref[...] = jnp.zeros_like(carry_ref)

    x = x_ref[...]
    wg = wg_ref[...]
    logits = jnp.dot(x, wg, preferred_element_type=jnp.float32)
    lt = logits.T

    iota_e = lax.broadcasted_iota(jnp.int32, (_E, _CT), 0)
    m1 = jnp.max(lt, axis=0, keepdims=True)
    i1 = jnp.min(jnp.where(lt == m1, iota_e, _E), axis=0, keepdims=True)
    oh1 = iota_e == i1
    masked = jnp.where(oh1, -jnp.inf, lt)
    m2 = jnp.max(masked, axis=0, keepdims=True)
    i2 = jnp.min(jnp.where(masked == m2, iota_e, _E), axis=0, keepdims=True)
    oh2 = iota_e == i2

    g1 = 1.0 / (1.0 + jnp.exp(m2 - m1))
    g2 = 1.0 - g1

    ohsum = (oh1 | oh2).astype(jnp.float32)
    ri = lax.broadcasted_iota(jnp.int32, (_CT, _CT), 0)
    cj = lax.broadcasted_iota(jnp.int32, (_CT, _CT), 1)
    sut = (ri < cj).astype(jnp.float32)
    cex = jnp.dot(ohsum, sut, preferred_element_type=jnp.float32)
    base = carry_ref[0:_E, 0:1]
    p = cex + base
    pos1 = jnp.sum(jnp.where(oh1, p, 0.0), axis=0)
    pos2 = jnp.sum(jnp.where(oh2, p, 0.0), axis=0)
    new_counts = base + jnp.sum(ohsum, axis=1, keepdims=True)
    carry_ref[0:_E, 0:1] = new_counts
    counts_ref[...] = new_counts.astype(jnp.int32)

    keep1 = pos1 < _CAP
    keep2 = pos2 < _CAP
    e1 = i1[0]
    e2 = i2[0]
    loc0_ref[...] = jnp.where(keep1, e1 * _CAP + pos1.astype(jnp.int32),
                              _PAD_ROW)
    loc1_ref[...] = jnp.where(keep2, e2 * _CAP + pos2.astype(jnp.int32),
                              _PAD_ROW)
    w0_ref[...] = jnp.where(keep1, g1[0], 0.0)
    w1_ref[...] = jnp.where(keep2, g2[0], 0.0)


_router = pl.pallas_call(
    _router_body,
    grid=(_G,),
    in_specs=[
        pl.BlockSpec((_CT, _D), lambda i: (i, 0)),
        pl.BlockSpec((_D, _E), lambda i: (0, 0)),
    ],
    out_specs=[
        pl.BlockSpec((_CT,), lambda i: (i,)),
        pl.BlockSpec((_CT,), lambda i: (i,)),
        pl.BlockSpec((_CT,), lambda i: (i,)),
        pl.BlockSpec((_CT,), lambda i: (i,)),
        pl.BlockSpec((_E, 1), lambda i: (0, 0)),
    ],
    out_shape=[
        jax.ShapeDtypeStruct((_T,), jnp.int32),
        jax.ShapeDtypeStruct((_T,), jnp.int32),
        jax.ShapeDtypeStruct((_T,), jnp.float32),
        jax.ShapeDtypeStruct((_T,), jnp.float32),
        jax.ShapeDtypeStruct((_E, 1), jnp.int32),
    ],
    scratch_shapes=[pltpu.VMEM((_E, 128), jnp.float32)],
)


def _dispatch_body(xt_hbm, loc0_hbm, loc1_hbm, buf_hbm,
                   idx0a_v, idx0b_v, idx1a_v, idx1b_v, rows0_v, rows1_v,
                   csem, ssem0, ssem1):
    wid = lax.axis_index("s") * _NC + lax.axis_index("c")
    base = wid * _TPW
    nch = _TPW // _CH
    idx0 = (idx0a_v, idx0b_v)
    idx1 = (idx1a_v, idx1b_v)
    rows = (rows0_v, rows1_v)
    ssem = (ssem0, ssem1)

    def load(ci, p):
        tok0 = base + ci * _CH
        pltpu.sync_copy(loc0_hbm.at[pl.ds(tok0, _CH)], idx0[p])
        pltpu.sync_copy(loc1_hbm.at[pl.ds(tok0, _CH)], idx1[p])
        return pltpu.async_copy(xt_hbm.at[pl.ds(tok0, _CH)], rows[p], csem)

    cp = load(0, 0)
    sc_prev = None
    for ci in range(nch):
        p = ci & 1
        cp.wait()
        s0 = pltpu.async_copy(rows[p], buf_hbm.at[idx0[p]], ssem[p])
        s1 = pltpu.async_copy(rows[p], buf_hbm.at[idx1[p]], ssem[p])
        if sc_prev is not None:
            sc_prev[0].wait()
            sc_prev[1].wait()
        if ci + 1 < nch:
            cp = load(ci + 1, 1 - p)
        sc_prev = (s0, s1)
    sc_prev[0].wait()
    sc_prev[1].wait()


@functools.lru_cache(maxsize=None)
def _get_dispatch():
    return functools.partial(
        pl.kernel,
        out_type=jax.ShapeDtypeStruct((_NROWS, _D), jnp.float32),
        mesh=plsc.VectorSubcoreMesh(core_axis_name="c", subcore_axis_name="s",
                                    num_cores=_NC, num_subcores=_NS),
        scratch_types=[
            pltpu.VMEM((_CH,), jnp.int32),
            pltpu.VMEM((_CH,), jnp.int32),
            pltpu.VMEM((_CH,), jnp.int32),
            pltpu.VMEM((_CH,), jnp.int32),
            pltpu.VMEM((_CH, _D), jnp.float32),
            pltpu.VMEM((_CH, _D), jnp.float32),
            pltpu.SemaphoreType.DMA,
            pltpu.SemaphoreType.DMA,
            pltpu.SemaphoreType.DMA,
        ],
        compiler_params=pltpu.CompilerParams(needs_layout_passes=False),
    )(_dispatch_body)


def _ffn_body(counts_ref, buf_ref, w1_ref, b1_ref, w2_ref, b2_ref, y_ref):
    e = pl.program_id(0)
    i = pl.program_id(1)
    live = jnp.where(e == _E, i == 0,
                     i * _RB < counts_ref[jnp.minimum(e, _E - 1)])

    @pl.when(live)
    def _():
        w1 = w1_ref[0]
        w2 = w2_ref[0]
        qr = _RB // 4
        hs = []
        for q in range(4):
            xq = buf_ref[q * qr:(q + 1) * qr, :]
            hs.append(jnp.dot(xq, w1, preferred_element_type=jnp.float32)
                      + b1_ref[0])
        for q in range(4):
            gq = jax.nn.gelu(hs[q])
            yq = jnp.dot(gq, w2, preferred_element_type=jnp.float32) + b2_ref[0]
            y_ref[q * qr:(q + 1) * qr, :] = yq


_BPE = _CAP // _RB


def _row_block(e, i):
    return jnp.minimum(e * _BPE + i, _NROWS // _RB - 1)


def _wexpert(e):
    return jnp.minimum(e, _E - 1)


_ffn = pl.pallas_call(
    _ffn_body,
    grid_spec=pltpu.PrefetchScalarGridSpec(
        num_scalar_prefetch=1,
        grid=(_E + 1, _BPE),
        in_specs=[
            pl.BlockSpec((_RB, _D), lambda e, i, c: (_row_block(e, i), 0)),
            pl.BlockSpec((1, _D, _DFF), lambda e, i, c: (_wexpert(e), 0, 0)),
            pl.BlockSpec((1, 1, _DFF), lambda e, i, c: (_wexpert(e), 0, 0)),
            pl.BlockSpec((1, _DFF, _D), lambda e, i, c: (_wexpert(e), 0, 0)),
            pl.BlockSpec((1, 1, _D), lambda e, i, c: (_wexpert(e), 0, 0)),
        ],
        out_specs=pl.BlockSpec((_RB, _D), lambda e, i, c: (_row_block(e, i), 0)),
    ),
    out_shape=jax.ShapeDtypeStruct((_NROWS, _D), jnp.float32),
)


_CCH = 32


def _combine_body(y_hbm, loc0_hbm, loc1_hbm, w0_hbm, w1_hbm, out_hbm,
                  idx0a_v, idx0b_v, idx1a_v, idx1b_v,
                  w0a_v, w0b_v, w1a_v, w1b_v,
                  a0_v, a1_v, b0_v, b1_v, gsem0, gsem1, osem):
    wid = lax.axis_index("s") * _NC + lax.axis_index("c")
    base = wid * _TPW
    nch = _TPW // _CCH
    idx0 = (idx0a_v, idx0b_v)
    idx1 = (idx1a_v, idx1b_v)
    w0 = (w0a_v, w0b_v)
    w1 = (w1a_v, w1b_v)
    av = (a0_v, a1_v)
    bv = (b0_v, b1_v)
    gsem = (gsem0, gsem1)

    def load(ci, p):
        tok0 = base + ci * _CCH
        pltpu.sync_copy(loc0_hbm.at[pl.ds(tok0, _CCH)], idx0[p])
        pltpu.sync_copy(loc1_hbm.at[pl.ds(tok0, _CCH)], idx1[p])
        pltpu.sync_copy(w0_hbm.at[pl.ds(tok0, _CCH)], w0[p])
        pltpu.sync_copy(w1_hbm.at[pl.ds(tok0, _CCH)], w1[p])
        ga = pltpu.async_copy(y_hbm.at[idx0[p]], av[p], gsem[p])
        gb = pltpu.async_copy(y_hbm.at[idx1[p]], bv[p], gsem[p])
        return ga, gb

    gs = [load(0, 0), load(1, 1)]
    for ci in range(nch):
        p = ci & 1
        gs[p][0].wait()
        gs[p][1].wait()

        a_v, b_v, w0_v, w1_v = av[p], bv[p], w0[p], w1[p]

        def tbody(t, _):
            tv = jnp.full((_LANES,), t, jnp.int32)
            wv0 = plsc.load_gather(w0_v, [tv])
            wv1 = plsc.load_gather(w1_v, [tv])
            for f in range(_D // _LANES):
                sl = pl.ds(f * _LANES, _LANES)
                a_v[t, sl] = a_v[t, sl] * wv0 + b_v[t, sl] * wv1
            return 0

        lax.fori_loop(0, _CCH, tbody, 0)
        oc = pltpu.async_copy(
            a_v, out_hbm.at[pl.ds(base + ci * _CCH, _CCH)], osem)
        oc.wait()
        if ci + 2 < nch:
            gs[p] = load(ci + 2, p)


@functools.lru_cache(maxsize=None)
def _get_combine():
    return functools.partial(
        pl.kernel,
        out_type=jax.ShapeDtypeStruct((_T, _D), jnp.float32),
        mesh=plsc.VectorSubcoreMesh(core_axis_name="c", subcore_axis_name="s",
                                    num_cores=_NC, num_subcores=_NS),
        scratch_types=(
            [pltpu.VMEM((_CCH,), jnp.int32)] * 4
            + [pltpu.VMEM((_CCH,), jnp.float32)] * 4
            + [pltpu.VMEM((_CCH, _D), jnp.float32)] * 4
            + [pltpu.SemaphoreType.DMA] * 3
        ),
        compiler_params=pltpu.CompilerParams(needs_layout_passes=False),
    )(_combine_body)


def kernel(x, Wg, W1, b1, W2, b2):
    xt = x.reshape(_T, _D)
    loc0, loc1, w0, w1, counts = _router(xt, Wg)
    buf = _get_dispatch()(xt, loc0, loc1)
    y = _ffn(counts.reshape(_E), buf, W1, b1.reshape(_E, 1, _DFF),
             W2, b2.reshape(_E, 1, _D))
    out = _get_combine()(y, loc0, loc1, w0, w1)
    return out.reshape(_B, _S, _D)
